# Initial kernel scaffold; baseline (speedup 1.0000x reference)
import jax
import jax.numpy as jnp
from jax.experimental import pallas as pl


def kernel(x, pos, edge_index, expert_weights, gate_w, gate_b, w1, w2):
    raise NotImplementedError("write your pallas kernel here")


# speedup vs baseline: 1.8265x; 1.8265x over previous
import functools
import jax
import jax.numpy as jnp
from jax.experimental import pallas as pl
from jax.experimental.pallas import tpu as pltpu

N = 10000
E = 320000
IN = 128
OUT = 128
K = 8
TOPK = 2
DIM = 2
NEG = -3.4028235e38


def _mm_kernel(x_ref, w_ref, o_ref):
    o_ref[...] = jnp.dot(x_ref[...], w_ref[...],
                         preferred_element_type=jnp.float32)


def _matmul(x, w, bn):
    n = x.shape[0]
    return pl.pallas_call(
        _mm_kernel,
        grid=(n // bn,),
        in_specs=[pl.BlockSpec((bn, x.shape[1]), lambda i: (i, 0)),
                  pl.BlockSpec((x.shape[1], w.shape[1]), lambda i: (0, 0))],
        out_specs=pl.BlockSpec((bn, w.shape[1]), lambda i: (i, 0)),
        out_shape=jax.ShapeDtypeStruct((n, w.shape[1]), jnp.float32),
    )(x, w)


def _mlp_kernel(p_ref, w1_ref, w2_ref, o_ref):
    p = p_ref[...]
    h = jnp.maximum(jnp.dot(p, w1_ref[...], preferred_element_type=jnp.float32), 0.0)
    o_ref[...] = jnp.dot(h, w2_ref[...], preferred_element_type=jnp.float32) + p


def _mlp(pooled, w1t, w2t, bn):
    n = pooled.shape[0]
    return pl.pallas_call(
        _mlp_kernel,
        grid=(n // bn,),
        in_specs=[pl.BlockSpec((bn, OUT), lambda i: (i, 0)),
                  pl.BlockSpec((OUT, 2 * OUT), lambda i: (0, 0)),
                  pl.BlockSpec((2 * OUT, OUT), lambda i: (0, 0))],
        out_specs=pl.BlockSpec((bn, OUT), lambda i: (i, 0)),
        out_shape=jax.ShapeDtypeStruct((n, OUT), jnp.float32),
    )(pooled, w1t, w2t)


def kernel(x, pos, edge_index, expert_weights, gate_w, gate_b, w1, w2):
    p = pos[:, :DIM]
    src = edge_index[:, 0]
    dst = edge_index[:, 1]
    diff_pos = jnp.take(p, dst, axis=0) - jnp.take(p, src, axis=0)
    gate_logits = diff_pos @ gate_w.T + gate_b
    _, topk_idx = jax.lax.top_k(gate_logits, TOPK)

    w_fused = jnp.transpose(expert_weights, (1, 0, 2)).reshape(IN, K * OUT)
    y = _matmul(x, w_fused, 1000).reshape(N, K, OUT)

    rows = dst[:, None] * K + topk_idx
    yy = y.reshape(N * K, OUT)
    msg = jnp.take(yy, rows[:, 0], axis=0) + jnp.take(yy, rows[:, 1], axis=0)

    pooled = jax.ops.segment_max(msg, src, num_segments=N)
    return _mlp(pooled, w1.T, w2.T, 1000)


# speedup vs baseline: 2.6366x; 1.4436x over previous
import functools
import jax
import jax.numpy as jnp
from jax import lax
from jax.experimental import pallas as pl
from jax.experimental.pallas import tpu as pltpu
from jax.experimental.pallas import tpu_sc as plsc

N = 10000
E = 320000
IN = 128
OUT = 128
K = 8
DIM = 2
NEG = -3.4028235e38

L = 16
NTILES = 32
NRANGE = 4
NSLICE = 8
NPR = N // NRANGE
EPT = E // NTILES
CH1 = 2000
CH2 = 2000
GB = 128
MBUF = CH2 + 2 * GB

_mesh = plsc.VectorSubcoreMesh(core_axis_name="c", subcore_axis_name="s")


def _iota():
    return lax.iota(jnp.int32, L)


def _splat_i32(v):
    return jnp.full((L,), v, dtype=jnp.int32)


def _bf16r(x):
    r = plsc.bitcast(x, jnp.int32)
    r2 = (r + 0x7FFF + (lax.shift_right_logical(r, 16) & 1)) & jnp.int32(-65536)
    return plsc.bitcast(r2, jnp.float32)


def _bcast_lane(vec, j):
    return vec.at[_splat_i32(j)].get(mode="promise_in_bounds")


@functools.partial(
    pl.kernel,
    out_type=(jax.ShapeDtypeStruct((E,), jnp.int32),
              jax.ShapeDtypeStruct((E,), jnp.int32),
              jax.ShapeDtypeStruct((E,), jnp.int32)),
    mesh=_mesh,
    compiler_params=pltpu.CompilerParams(needs_layout_passes=False),
    scratch_types=[
        pltpu.VMEM((N * DIM,), jnp.float32),
        pltpu.VMEM((128,), jnp.float32),
        pltpu.VMEM((CH1 * 2,), jnp.int32),
        pltpu.VMEM((CH1,), jnp.int32),
        pltpu.VMEM((CH1,), jnp.int32),
        pltpu.VMEM((CH1,), jnp.int32),
    ],
)
def _route(pos_hbm, ei_hbm, gp_hbm, src_out, id1_out, id2_out,
           pos_v, gp_v, ei_v, so_v, i1_v, i2_v):
    wid = lax.axis_index("s") * 2 + lax.axis_index("c")
    base = wid * EPT
    pltpu.sync_copy(pos_hbm, pos_v)
    pltpu.sync_copy(gp_hbm, gp_v)
    iota = _iota()
    g0 = gp_v[pl.ds(0, L)]
    g1 = gp_v[pl.ds(L, L)]
    gwx = [_bcast_lane(g0, 2 * k) for k in range(K)]
    gwy = [_bcast_lane(g0, 2 * k + 1) for k in range(K)]
    gb = [_bcast_lane(g1, k) for k in range(K)]

    def chunk_body(ch, _):
        off = base + ch * CH1
        pltpu.sync_copy(ei_hbm.at[pl.ds(off * 2, CH1 * 2)], ei_v)

        def vec_body(i, _):
            b2 = i * (2 * L)
            src = plsc.load_gather(ei_v, [iota * 2 + b2])
            dst = plsc.load_gather(ei_v, [iota * 2 + b2 + 1])
            pxs = plsc.load_gather(pos_v, [src * 2])
            pys = plsc.load_gather(pos_v, [src * 2 + 1])
            pxd = plsc.load_gather(pos_v, [dst * 2])
            pyd = plsc.load_gather(pos_v, [dst * 2 + 1])
            dx = _bf16r(pxd - pxs)
            dy = _bf16r(pyd - pys)
            lg = [dx * gwx[k] + dy * gwy[k] + gb[k] for k in range(K)]
            m1 = lg[0]
            for k in range(1, K):
                m1 = jnp.maximum(m1, lg[k])
            a1 = _splat_i32(K)
            for k in range(K):
                a1 = jnp.minimum(a1, jnp.where(lg[k] == m1, _splat_i32(k), K))
            m2 = jnp.full((L,), NEG, jnp.float32)
            for k in range(K):
                lgk = jnp.where(a1 == k, NEG, lg[k])
                m2 = jnp.maximum(m2, lgk)
            a2 = _splat_i32(K)
            for k in range(K):
                hit = (lg[k] == m2) & (a1 != k)
                a2 = jnp.minimum(a2, jnp.where(hit, _splat_i32(k), K))
            so_v[pl.ds(i * L, L)] = src
            i1_v[pl.ds(i * L, L)] = dst * (K * NSLICE) + a1 * NSLICE
            i2_v[pl.ds(i * L, L)] = dst * (K * NSLICE) + a2 * NSLICE
            return 0

        lax.fori_loop(0, CH1 // L, vec_body, 0)
        pltpu.sync_copy(so_v, src_out.at[pl.ds(off, CH1)])
        pltpu.sync_copy(i1_v, id1_out.at[pl.ds(off, CH1)])
        pltpu.sync_copy(i2_v, id2_out.at[pl.ds(off, CH1)])
        return 0

    lax.fori_loop(0, EPT // CH1, chunk_body, 0)


@functools.partial(
    pl.kernel,
    out_type=jax.ShapeDtypeStruct((NSLICE * N * L,), jnp.float32),
    mesh=_mesh,
    compiler_params=pltpu.CompilerParams(needs_layout_passes=False, use_tc_tiling_on_sc=False),
    scratch_types=[
        pltpu.VMEM(((NPR + 8) * L,), jnp.float32),
        pltpu.VMEM((CH2,), jnp.int32),
        pltpu.VMEM((CH2,), jnp.int32),
        pltpu.VMEM((CH2,), jnp.int32),
        pltpu.VMEM((MBUF,), jnp.int32),
        pltpu.VMEM((MBUF,), jnp.int32),
        pltpu.VMEM((MBUF,), jnp.int32),
        pltpu.VMEM((GB, L), jnp.float32),
        pltpu.VMEM((GB, L), jnp.float32),
        pltpu.SemaphoreType.DMA,
        pltpu.SemaphoreType.DMA,
    ],
)
def _segmax(src_hbm, id1_hbm, id2_hbm, y_hbm, out_hbm,
            acc_v, src_v, id1_v, id2_v, ms_v, m1_v, m2_v,
            r1_v, r2_v, sem1, sem2):
    wid = lax.axis_index("s") * 2 + lax.axis_index("c")
    r = wid // NSLICE
    s = wid % NSLICE
    lo = r * NPR
    iota = _iota()
    negv = jnp.full((L,), NEG, jnp.float32)

    def init_body(i, _):
        acc_v[pl.ds(i * L, L)] = negv
        return 0

    lax.fori_loop(0, NPR + 8, init_body, 0)

    def zero_body(i, _):
        m1_v[pl.ds(i * L, L)] = _splat_i32(0)
        m2_v[pl.ds(i * L, L)] = _splat_i32(0)
        return 0

    lax.fori_loop(0, MBUF // L, zero_body, 0)

    def chunk_body(ch, _):
        off = ch * CH2
        pltpu.sync_copy(src_hbm.at[pl.ds(off, CH2)], src_v)
        pltpu.sync_copy(id1_hbm.at[pl.ds(off, CH2)], id1_v)
        pltpu.sync_copy(id2_hbm.at[pl.ds(off, CH2)], id2_v)

        def scan_body(i, p):
            sv = src_v[pl.ds(i * L, L)]
            m = (sv >= lo) & (sv < lo + NPR)
            i1 = id1_v[pl.ds(i * L, L)] + s
            i2 = id2_v[pl.ds(i * L, L)] + s
            plsc.store_compressed(ms_v.at[pl.ds(p, L)], sv, mask=m)
            plsc.store_compressed(m1_v.at[pl.ds(p, L)], i1, mask=m)
            plsc.store_compressed(m2_v.at[pl.ds(p, L)], i2, mask=m)
            return p + jnp.sum(m.astype(jnp.int32))

        p = lax.fori_loop(0, CH2 // L, scan_body, 0)

        dummy = _splat_i32(lo + NPR)
        for t in range(GB // L):
            ms_v[pl.ds(p + t * L, L)] = dummy
            m1_v[pl.ds(p + t * L, L)] = _splat_i32(0)
            m2_v[pl.ds(p + t * L, L)] = _splat_i32(0)

        def batch_body(b, _):
            pltpu.async_copy(y_hbm.at[m1_v.at[pl.ds(b * GB, GB)]], r1_v, sem1)
            pltpu.async_copy(y_hbm.at[m2_v.at[pl.ds(b * GB, GB)]], r2_v, sem2).wait()
            pltpu.make_async_copy(y_hbm.at[m1_v.at[pl.ds(b * GB, GB)]], r1_v, sem1).wait()

            def grp_body(g, _):
                sv = ms_v[pl.ds(b * GB + g * L, L)]
                for j in range(L):
                    sj = _bcast_lane(sv, j)
                    ia = (sj - lo) * L + iota
                    rown = _splat_i32(g * L + j)
                    msg = (plsc.load_gather(r1_v, [rown, iota])
                           + plsc.load_gather(r2_v, [rown, iota]))
                    a = plsc.load_gather(acc_v, [ia])
                    plsc.store_scatter(acc_v, [ia], jnp.maximum(a, msg))
                return 0

            lax.fori_loop(0, GB // L, grp_body, 0)
            return 0

        nb = (p + GB - 1) // GB
        lax.fori_loop(0, nb, batch_body, 0)
        return 0

    lax.fori_loop(0, E // CH2, chunk_body, 0)
    pltpu.sync_copy(acc_v.at[pl.ds(0, NPR * L)],
                    out_hbm.at[pl.ds((s * N + lo) * L, NPR * L)])


def _mm_kernel(x_ref, w_ref, o_ref):
    o_ref[...] = jnp.dot(x_ref[...], w_ref[...],
                         preferred_element_type=jnp.float32)


def _matmul(x, w, bn):
    n = x.shape[0]
    return pl.pallas_call(
        _mm_kernel,
        grid=(n // bn,),
        in_specs=[pl.BlockSpec((bn, x.shape[1]), lambda i: (i, 0)),
                  pl.BlockSpec((x.shape[1], w.shape[1]), lambda i: (0, 0))],
        out_specs=pl.BlockSpec((bn, w.shape[1]), lambda i: (i, 0)),
        out_shape=jax.ShapeDtypeStruct((n, w.shape[1]), jnp.float32),
    )(x, w)


def _mlp_kernel(p_ref, w1_ref, w2_ref, o_ref):
    p = p_ref[...]
    h = jnp.maximum(jnp.dot(p, w1_ref[...], preferred_element_type=jnp.float32), 0.0)
    o_ref[...] = jnp.dot(h, w2_ref[...], preferred_element_type=jnp.float32) + p


def _mlp(pooled, w1t, w2t, bn):
    n = pooled.shape[0]
    return pl.pallas_call(
        _mlp_kernel,
        grid=(n // bn,),
        in_specs=[pl.BlockSpec((bn, OUT), lambda i: (i, 0)),
                  pl.BlockSpec((OUT, 2 * OUT), lambda i: (0, 0)),
                  pl.BlockSpec((2 * OUT, OUT), lambda i: (0, 0))],
        out_specs=pl.BlockSpec((bn, OUT), lambda i: (i, 0)),
        out_shape=jax.ShapeDtypeStruct((n, OUT), jnp.float32),
    )(pooled, w1t, w2t)


def kernel(x, pos, edge_index, expert_weights, gate_w, gate_b, w1, w2):
    gwi = lax.bitcast_convert_type(gate_w, jnp.int32)
    gwi = (gwi + 0x7FFF + (lax.shift_right_logical(gwi, 16) & 1)) & jnp.int32(-65536)
    gwr = lax.bitcast_convert_type(gwi, jnp.float32)
    gp = jnp.concatenate([gwr.reshape(-1), gate_b,
                          jnp.zeros((104,), jnp.float32)])
    src_a, id1_a, id2_a = _route(pos[:, :DIM].reshape(-1),
                                 edge_index.reshape(-1), gp)

    w_fused = jnp.transpose(expert_weights, (1, 0, 2)).reshape(IN, K * OUT)
    y = _matmul(x, w_fused, 1000)

    pooled_t = _segmax(src_a, id1_a, id2_a, y.reshape(N * K * NSLICE, L))
    pooled = jnp.transpose(pooled_t.reshape(NSLICE, N, L),
                           (1, 0, 2)).reshape(N, OUT)
    return _mlp(pooled, w1.T, w2.T, 1000)


# speedup vs baseline: 3.9798x; 1.5094x over previous
import functools
import jax
import jax.numpy as jnp
from jax import lax
from jax.experimental import pallas as pl
from jax.experimental.pallas import tpu as pltpu
from jax.experimental.pallas import tpu_sc as plsc

N = 10000
E = 320000
IN = 128
OUT = 128
K = 8
DIM = 2
NEG = -3.4028235e38

L = 16
NTILES = 32
NRANGE = 4
NSLICE = 8
NPR = N // NRANGE
EPT = E // NTILES
CH1 = 2000
CH2 = 8000
GB = 128
SB = 512
MBUF = CH2 + 2 * SB

_mesh = plsc.VectorSubcoreMesh(core_axis_name="c", subcore_axis_name="s")


def _iota():
    return lax.iota(jnp.int32, L)


def _splat_i32(v):
    return jnp.full((L,), v, dtype=jnp.int32)


def _bf16r(x):
    r = plsc.bitcast(x, jnp.int32)
    r2 = (r + 0x7FFF + (lax.shift_right_logical(r, 16) & 1)) & jnp.int32(-65536)
    return plsc.bitcast(r2, jnp.float32)


def _bcast_lane(vec, j):
    return vec.at[_splat_i32(j)].get(mode="promise_in_bounds")


@functools.partial(
    pl.kernel,
    out_type=(jax.ShapeDtypeStruct((E,), jnp.int32),
              jax.ShapeDtypeStruct((E,), jnp.int32),
              jax.ShapeDtypeStruct((E,), jnp.int32)),
    mesh=_mesh,
    compiler_params=pltpu.CompilerParams(needs_layout_passes=False),
    scratch_types=[
        pltpu.VMEM((N * DIM,), jnp.float32),
        pltpu.VMEM((128,), jnp.float32),
        pltpu.VMEM((CH1 * 2,), jnp.int32),
        pltpu.VMEM((CH1,), jnp.int32),
        pltpu.VMEM((CH1,), jnp.int32),
        pltpu.VMEM((CH1,), jnp.int32),
    ],
)
def _route(pos_hbm, ei_hbm, gp_hbm, src_out, id1_out, id2_out,
           pos_v, gp_v, ei_v, so_v, i1_v, i2_v):
    wid = lax.axis_index("s") * 2 + lax.axis_index("c")
    base = wid * EPT
    pltpu.sync_copy(pos_hbm, pos_v)
    pltpu.sync_copy(gp_hbm, gp_v)
    iota = _iota()
    g0 = gp_v[pl.ds(0, L)]
    g1 = gp_v[pl.ds(L, L)]
    gwx = [_bcast_lane(g0, 2 * k) for k in range(K)]
    gwy = [_bcast_lane(g0, 2 * k + 1) for k in range(K)]
    gb = [_bcast_lane(g1, k) for k in range(K)]

    def chunk_body(ch, _):
        off = base + ch * CH1
        pltpu.sync_copy(ei_hbm.at[pl.ds(off * 2, CH1 * 2)], ei_v)

        def vec_body(i, _):
            b2 = i * (2 * L)
            src = plsc.load_gather(ei_v, [iota * 2 + b2])
            dst = plsc.load_gather(ei_v, [iota * 2 + b2 + 1])
            pxs = plsc.load_gather(pos_v, [src * 2])
            pys = plsc.load_gather(pos_v, [src * 2 + 1])
            pxd = plsc.load_gather(pos_v, [dst * 2])
            pyd = plsc.load_gather(pos_v, [dst * 2 + 1])
            dx = _bf16r(pxd - pxs)
            dy = _bf16r(pyd - pys)
            lg = [dx * gwx[k] + dy * gwy[k] + gb[k] for k in range(K)]
            m1 = lg[0]
            for k in range(1, K):
                m1 = jnp.maximum(m1, lg[k])
            a1 = _splat_i32(K)
            for k in range(K):
                a1 = jnp.minimum(a1, jnp.where(lg[k] == m1, _splat_i32(k), K))
            m2 = jnp.full((L,), NEG, jnp.float32)
            for k in range(K):
                lgk = jnp.where(a1 == k, NEG, lg[k])
                m2 = jnp.maximum(m2, lgk)
            a2 = _splat_i32(K)
            for k in range(K):
                hit = (lg[k] == m2) & (a1 != k)
                a2 = jnp.minimum(a2, jnp.where(hit, _splat_i32(k), K))
            so_v[pl.ds(i * L, L)] = src
            i1_v[pl.ds(i * L, L)] = dst * (K * NSLICE) + a1 * NSLICE
            i2_v[pl.ds(i * L, L)] = dst * (K * NSLICE) + a2 * NSLICE
            return 0

        lax.fori_loop(0, CH1 // L, vec_body, 0)
        pltpu.sync_copy(so_v, src_out.at[pl.ds(off, CH1)])
        pltpu.sync_copy(i1_v, id1_out.at[pl.ds(off, CH1)])
        pltpu.sync_copy(i2_v, id2_out.at[pl.ds(off, CH1)])
        return 0

    lax.fori_loop(0, EPT // CH1, chunk_body, 0)


@functools.partial(
    pl.kernel,
    out_type=jax.ShapeDtypeStruct((NSLICE * N * L,), jnp.float32),
    mesh=_mesh,
    compiler_params=pltpu.CompilerParams(needs_layout_passes=False, use_tc_tiling_on_sc=False),
    scratch_types=[
        pltpu.VMEM(((NPR + 8) * L,), jnp.float32),
        pltpu.VMEM((CH2,), jnp.int32),
        pltpu.VMEM((CH2,), jnp.int32),
        pltpu.VMEM((CH2,), jnp.int32),
        pltpu.VMEM((MBUF,), jnp.int32),
        pltpu.VMEM((MBUF,), jnp.int32),
        pltpu.VMEM((MBUF,), jnp.int32),
        pltpu.VMEM((SB, L), jnp.float32),
        pltpu.VMEM((SB, L), jnp.float32),
        pltpu.VMEM((SB, L), jnp.float32),
        pltpu.VMEM((SB, L), jnp.float32),
        pltpu.SemaphoreType.DMA,
        pltpu.SemaphoreType.DMA,
        pltpu.SemaphoreType.DMA,
        pltpu.SemaphoreType.DMA,
        pltpu.SemaphoreType.DMA,
    ],
)
def _segmax(src_hbm, id1_hbm, id2_hbm, y_hbm, out_hbm,
            acc_v, src_v, id1_v, id2_v, ms_v, m1_v, m2_v,
            r1a_v, r2a_v, r1b_v, r2b_v, semA, semB, semc1, semc2, semc3):
    wid = lax.axis_index("s") * 2 + lax.axis_index("c")
    r = wid // NSLICE
    s = wid % NSLICE
    lo = r * NPR
    iota = _iota()
    negv = jnp.full((L,), NEG, jnp.float32)

    def init_body(i, _):
        acc_v[pl.ds(i * L, L)] = negv
        return 0

    lax.fori_loop(0, NPR + 8, init_body, 0)

    def zero_body(i, _):
        m1_v[pl.ds(i * L, L)] = _splat_i32(0)
        m2_v[pl.ds(i * L, L)] = _splat_i32(0)
        return 0

    lax.fori_loop(0, MBUF // L, zero_body, 0)

    NCH = E // CH2

    def issue_chunk(ch):
        off = ch * CH2
        pltpu.async_copy(src_hbm.at[pl.ds(off, CH2)], src_v, semc1)
        pltpu.async_copy(id1_hbm.at[pl.ds(off, CH2)], id1_v, semc2)
        pltpu.async_copy(id2_hbm.at[pl.ds(off, CH2)], id2_v, semc3)

    def wait_chunk():
        pltpu.make_async_copy(src_hbm.at[pl.ds(0, CH2)], src_v, semc1).wait()
        pltpu.make_async_copy(id1_hbm.at[pl.ds(0, CH2)], id1_v, semc2).wait()
        pltpu.make_async_copy(id2_hbm.at[pl.ds(0, CH2)], id2_v, semc3).wait()

    def issue_sb(t, r1buf, r2buf, sem):
        for q in range(SB // GB):
            toff = t * SB + q * GB
            pltpu.async_copy(y_hbm.at[m1_v.at[pl.ds(toff, GB)]],
                             r1buf.at[pl.ds(q * GB, GB)], sem)
            pltpu.async_copy(y_hbm.at[m2_v.at[pl.ds(toff, GB)]],
                             r2buf.at[pl.ds(q * GB, GB)], sem)

    def wait_sb(r1buf, r2buf, sem):
        for q in range(SB // GB):
            pltpu.make_async_copy(y_hbm.at[m1_v.at[pl.ds(0, GB)]],
                                  r1buf.at[pl.ds(q * GB, GB)], sem).wait()
            pltpu.make_async_copy(y_hbm.at[m2_v.at[pl.ds(0, GB)]],
                                  r2buf.at[pl.ds(q * GB, GB)], sem).wait()

    def rmw_sb(t, r1buf, r2buf):
        def grp_body(g, _):
            iab = ms_v[pl.ds(t * SB + g * L, L)]
            for j in range(L):
                ia = _bcast_lane(iab, j) + iota
                rown = _splat_i32(g * L + j)
                msg = (plsc.load_gather(r1buf, [rown, iota])
                       + plsc.load_gather(r2buf, [rown, iota]))
                a = plsc.load_gather(acc_v, [ia])
                plsc.store_scatter(acc_v, [ia], jnp.maximum(a, msg))
            return 0

        lax.fori_loop(0, SB // L, grp_body, 0)

    issue_chunk(0)

    def chunk_body(ch, _):
        wait_chunk()

        def scan_body(i, p):
            sv = src_v[pl.ds(i * L, L)]
            m = (sv >= lo) & (sv < lo + NPR)
            iab = (sv - lo) * L
            i1 = id1_v[pl.ds(i * L, L)] + s
            i2 = id2_v[pl.ds(i * L, L)] + s
            plsc.store_compressed(ms_v.at[pl.ds(p, L)], iab, mask=m)
            plsc.store_compressed(m1_v.at[pl.ds(p, L)], i1, mask=m)
            plsc.store_compressed(m2_v.at[pl.ds(p, L)], i2, mask=m)
            return p + jnp.sum(m.astype(jnp.int32))

        p = lax.fori_loop(0, CH2 // L, scan_body, 0)

        @pl.when(ch + 1 < NCH)
        def _():
            issue_chunk(ch + 1)

        for t in range(SB // L):
            ms_v[pl.ds(p + t * L, L)] = _splat_i32(NPR * L)
            m1_v[pl.ds(p + t * L, L)] = _splat_i32(0)
            m2_v[pl.ds(p + t * L, L)] = _splat_i32(0)

        nsb = (p + SB - 1) // SB

        @pl.when(nsb > 0)
        def _():
            issue_sb(0, r1a_v, r2a_v, semA)

        def pair_body(tt, _):
            t0 = tt * 2

            @pl.when(t0 < nsb)
            def _():
                @pl.when(t0 + 1 < nsb)
                def _():
                    issue_sb(t0 + 1, r1b_v, r2b_v, semB)

                wait_sb(r1a_v, r2a_v, semA)
                rmw_sb(t0, r1a_v, r2a_v)

            @pl.when(t0 + 1 < nsb)
            def _():
                @pl.when(t0 + 2 < nsb)
                def _():
                    issue_sb(t0 + 2, r1a_v, r2a_v, semA)

                wait_sb(r1b_v, r2b_v, semB)
                rmw_sb(t0 + 1, r1b_v, r2b_v)

            return 0

        lax.fori_loop(0, (nsb + 1) // 2, pair_body, 0)
        return 0

    lax.fori_loop(0, NCH, chunk_body, 0)
    pltpu.sync_copy(acc_v.at[pl.ds(0, NPR * L)],
                    out_hbm.at[pl.ds((s * N + lo) * L, NPR * L)])


def _mm_kernel(x_ref, w_ref, o_ref):
    o_ref[...] = jnp.dot(x_ref[...], w_ref[...],
                         preferred_element_type=jnp.float32)


def _matmul(x, w, bn):
    n = x.shape[0]
    return pl.pallas_call(
        _mm_kernel,
        grid=(n // bn,),
        in_specs=[pl.BlockSpec((bn, x.shape[1]), lambda i: (i, 0)),
                  pl.BlockSpec((x.shape[1], w.shape[1]), lambda i: (0, 0))],
        out_specs=pl.BlockSpec((bn, w.shape[1]), lambda i: (i, 0)),
        out_shape=jax.ShapeDtypeStruct((n, w.shape[1]), jnp.float32),
    )(x, w)


def _mlp_kernel(p_ref, w1_ref, w2_ref, o_ref):
    p = p_ref[...]
    h = jnp.maximum(jnp.dot(p, w1_ref[...], preferred_element_type=jnp.float32), 0.0)
    o_ref[...] = jnp.dot(h, w2_ref[...], preferred_element_type=jnp.float32) + p


def _mlp(pooled, w1t, w2t, bn):
    n = pooled.shape[0]
    return pl.pallas_call(
        _mlp_kernel,
        grid=(n // bn,),
        in_specs=[pl.BlockSpec((bn, OUT), lambda i: (i, 0)),
                  pl.BlockSpec((OUT, 2 * OUT), lambda i: (0, 0)),
                  pl.BlockSpec((2 * OUT, OUT), lambda i: (0, 0))],
        out_specs=pl.BlockSpec((bn, OUT), lambda i: (i, 0)),
        out_shape=jax.ShapeDtypeStruct((n, OUT), jnp.float32),
    )(pooled, w1t, w2t)


def kernel(x, pos, edge_index, expert_weights, gate_w, gate_b, w1, w2):
    gwi = lax.bitcast_convert_type(gate_w, jnp.int32)
    gwi = (gwi + 0x7FFF + (lax.shift_right_logical(gwi, 16) & 1)) & jnp.int32(-65536)
    gwr = lax.bitcast_convert_type(gwi, jnp.float32)
    gp = jnp.concatenate([gwr.reshape(-1), gate_b,
                          jnp.zeros((104,), jnp.float32)])
    src_a, id1_a, id2_a = _route(pos[:, :DIM].reshape(-1),
                                 edge_index.reshape(-1), gp)

    w_fused = jnp.transpose(expert_weights, (1, 0, 2)).reshape(IN, K * OUT)
    y = _matmul(x, w_fused, 1000)

    pooled_t = _segmax(src_a, id1_a, id2_a, y.reshape(N * K * NSLICE, L))
    pooled = jnp.transpose(pooled_t.reshape(NSLICE, N, L),
                           (1, 0, 2)).reshape(N, OUT)
    return _mlp(pooled, w1.T, w2.T, 1000)


# speedup vs baseline: 4.1252x; 1.0365x over previous
import functools
import jax
import jax.numpy as jnp
from jax import lax
from jax.experimental import pallas as pl
from jax.experimental.pallas import tpu as pltpu
from jax.experimental.pallas import tpu_sc as plsc

N = 10000
E = 320000
IN = 128
OUT = 128
K = 8
DIM = 2
NEG = -3.4028235e38

L = 16
NTILES = 32
NRANGE = 4
NSLICE = 8
NPR = N // NRANGE
EPT = E // NTILES
CH1 = 2000
CH2 = 8000
GB = 128
SB = 512
MBUF = CH2 + 2 * SB

_mesh = plsc.VectorSubcoreMesh(core_axis_name="c", subcore_axis_name="s")


def _iota():
    return lax.iota(jnp.int32, L)


def _splat_i32(v):
    return jnp.full((L,), v, dtype=jnp.int32)


def _bf16r(x):
    r = plsc.bitcast(x, jnp.int32)
    r2 = (r + 0x7FFF + (lax.shift_right_logical(r, 16) & 1)) & jnp.int32(-65536)
    return plsc.bitcast(r2, jnp.float32)


def _bcast_lane(vec, j):
    return vec.at[_splat_i32(j)].get(mode="promise_in_bounds")


@functools.partial(
    pl.kernel,
    out_type=(jax.ShapeDtypeStruct((E,), jnp.int32),
              jax.ShapeDtypeStruct((E,), jnp.int32),
              jax.ShapeDtypeStruct((E,), jnp.int32)),
    mesh=_mesh,
    compiler_params=pltpu.CompilerParams(needs_layout_passes=False),
    scratch_types=[
        pltpu.VMEM((N * DIM,), jnp.float32),
        pltpu.VMEM((128,), jnp.float32),
        pltpu.VMEM((CH1 * 2,), jnp.int32),
        pltpu.VMEM((CH1,), jnp.int32),
        pltpu.VMEM((CH1,), jnp.int32),
        pltpu.VMEM((CH1,), jnp.int32),
    ],
)
def _route(pos_hbm, ei_hbm, gp_hbm, src_out, id1_out, id2_out,
           pos_v, gp_v, ei_v, so_v, i1_v, i2_v):
    wid = lax.axis_index("s") * 2 + lax.axis_index("c")
    base = wid * EPT
    pltpu.sync_copy(pos_hbm, pos_v)
    pltpu.sync_copy(gp_hbm, gp_v)
    iota = _iota()
    g0 = gp_v[pl.ds(0, L)]
    g1 = gp_v[pl.ds(L, L)]
    gwx = [_bcast_lane(g0, 2 * k) for k in range(K)]
    gwy = [_bcast_lane(g0, 2 * k + 1) for k in range(K)]
    gb = [_bcast_lane(g1, k) for k in range(K)]

    def chunk_body(ch, _):
        off = base + ch * CH1
        pltpu.sync_copy(ei_hbm.at[pl.ds(off * 2, CH1 * 2)], ei_v)

        def vec_body(i, _):
            b2 = i * (2 * L)
            src = plsc.load_gather(ei_v, [iota * 2 + b2])
            dst = plsc.load_gather(ei_v, [iota * 2 + b2 + 1])
            pxs = plsc.load_gather(pos_v, [src * 2])
            pys = plsc.load_gather(pos_v, [src * 2 + 1])
            pxd = plsc.load_gather(pos_v, [dst * 2])
            pyd = plsc.load_gather(pos_v, [dst * 2 + 1])
            dx = _bf16r(pxd - pxs)
            dy = _bf16r(pyd - pys)
            lg = [dx * gwx[k] + dy * gwy[k] + gb[k] for k in range(K)]
            m1 = lg[0]
            for k in range(1, K):
                m1 = jnp.maximum(m1, lg[k])
            a1 = _splat_i32(K)
            for k in range(K):
                a1 = jnp.minimum(a1, jnp.where(lg[k] == m1, _splat_i32(k), K))
            m2 = jnp.full((L,), NEG, jnp.float32)
            for k in range(K):
                lgk = jnp.where(a1 == k, NEG, lg[k])
                m2 = jnp.maximum(m2, lgk)
            a2 = _splat_i32(K)
            for k in range(K):
                hit = (lg[k] == m2) & (a1 != k)
                a2 = jnp.minimum(a2, jnp.where(hit, _splat_i32(k), K))
            so_v[pl.ds(i * L, L)] = src
            i1_v[pl.ds(i * L, L)] = dst * (K * NSLICE) + a1 * NSLICE
            i2_v[pl.ds(i * L, L)] = dst * (K * NSLICE) + a2 * NSLICE
            return 0

        lax.fori_loop(0, CH1 // L, vec_body, 0)
        pltpu.sync_copy(so_v, src_out.at[pl.ds(off, CH1)])
        pltpu.sync_copy(i1_v, id1_out.at[pl.ds(off, CH1)])
        pltpu.sync_copy(i2_v, id2_out.at[pl.ds(off, CH1)])
        return 0

    lax.fori_loop(0, EPT // CH1, chunk_body, 0)


@functools.partial(
    pl.kernel,
    out_type=jax.ShapeDtypeStruct((NSLICE * N * L,), jnp.float32),
    mesh=_mesh,
    compiler_params=pltpu.CompilerParams(needs_layout_passes=False, use_tc_tiling_on_sc=False),
    scratch_types=[
        pltpu.VMEM(((NPR + 8) * L,), jnp.float32),
        pltpu.VMEM((CH2,), jnp.int32),
        pltpu.VMEM((CH2,), jnp.int32),
        pltpu.VMEM((CH2,), jnp.int32),
        pltpu.VMEM((MBUF,), jnp.int32),
        pltpu.VMEM((MBUF,), jnp.int32),
        pltpu.VMEM((MBUF,), jnp.int32),
        pltpu.VMEM((SB, L), jnp.float32),
        pltpu.VMEM((SB, L), jnp.float32),
        pltpu.VMEM((SB, L), jnp.float32),
        pltpu.VMEM((SB, L), jnp.float32),
        pltpu.SemaphoreType.DMA,
        pltpu.SemaphoreType.DMA,
        pltpu.SemaphoreType.DMA,
        pltpu.SemaphoreType.DMA,
        pltpu.SemaphoreType.DMA,
    ],
)
def _segmax(src_hbm, id1_hbm, id2_hbm, y_hbm, out_hbm,
            acc_v, src_v, id1_v, id2_v, ms_v, m1_v, m2_v,
            r1a_v, r2a_v, r1b_v, r2b_v, semA, semB, semc1, semc2, semc3):
    wid = lax.axis_index("s") * 2 + lax.axis_index("c")
    r = wid // NSLICE
    s = wid % NSLICE
    lo = r * NPR
    iota = _iota()
    negv = jnp.full((L,), NEG, jnp.float32)

    def init_body(i, _):
        acc_v[pl.ds(i * L, L)] = negv
        return 0

    lax.fori_loop(0, NPR + 8, init_body, 0)

    def zero_body(i, _):
        m1_v[pl.ds(i * L, L)] = _splat_i32(0)
        m2_v[pl.ds(i * L, L)] = _splat_i32(0)
        return 0

    lax.fori_loop(0, MBUF // L, zero_body, 0)

    NCH = E // CH2

    def issue_chunk(ch):
        off = ch * CH2
        pltpu.async_copy(src_hbm.at[pl.ds(off, CH2)], src_v, semc1)
        pltpu.async_copy(id1_hbm.at[pl.ds(off, CH2)], id1_v, semc2)
        pltpu.async_copy(id2_hbm.at[pl.ds(off, CH2)], id2_v, semc3)

    def wait_chunk():
        pltpu.make_async_copy(src_hbm.at[pl.ds(0, CH2)], src_v, semc1).wait()
        pltpu.make_async_copy(id1_hbm.at[pl.ds(0, CH2)], id1_v, semc2).wait()
        pltpu.make_async_copy(id2_hbm.at[pl.ds(0, CH2)], id2_v, semc3).wait()

    def issue_sb(t, r1buf, r2buf, sem):
        for q in range(SB // GB):
            toff = t * SB + q * GB
            pltpu.async_copy(y_hbm.at[m1_v.at[pl.ds(toff, GB)]],
                             r1buf.at[pl.ds(q * GB, GB)], sem)
            pltpu.async_copy(y_hbm.at[m2_v.at[pl.ds(toff, GB)]],
                             r2buf.at[pl.ds(q * GB, GB)], sem)

    def wait_sb(r1buf, r2buf, sem):
        for q in range(SB // GB):
            pltpu.make_async_copy(y_hbm.at[m1_v.at[pl.ds(0, GB)]],
                                  r1buf.at[pl.ds(q * GB, GB)], sem).wait()
            pltpu.make_async_copy(y_hbm.at[m2_v.at[pl.ds(0, GB)]],
                                  r2buf.at[pl.ds(q * GB, GB)], sem).wait()

    def rmw_sb(t, r1buf, r2buf):
        def grp_body(g, _):
            iab = ms_v[pl.ds(t * SB + g * L, L)]
            for j in range(0, L, 4):
                b = [_bcast_lane(iab, j + u) for u in range(4)]
                msg = [plsc.load_gather(r1buf, [_splat_i32(g * L + j + u), iota])
                       + plsc.load_gather(r2buf, [_splat_i32(g * L + j + u), iota])
                       for u in range(4)]
                alive = [None] * 4
                for i in range(3):
                    dead = None
                    for k2 in range(i + 1, 4):
                        eq = b[i] == b[k2]
                        msg[k2] = jnp.where(eq, jnp.maximum(msg[k2], msg[i]),
                                            msg[k2])
                        dead = eq if dead is None else (dead | eq)
                    alive[i] = jnp.logical_not(dead)
                ia = [b[u] + iota for u in range(4)]
                mx = [jnp.maximum(plsc.load_gather(acc_v, [ia[u]]), msg[u])
                      for u in range(4)]
                for u in range(4):
                    if alive[u] is None:
                        plsc.store_scatter(acc_v, [ia[u]], mx[u])
                    else:
                        plsc.store_scatter(acc_v, [ia[u]], mx[u], mask=alive[u])
            return 0

        lax.fori_loop(0, SB // L, grp_body, 0)

    issue_chunk(0)

    def chunk_body(ch, _):
        wait_chunk()

        def scan_body(i, p):
            sv = src_v[pl.ds(i * L, L)]
            m = (sv >= lo) & (sv < lo + NPR)
            iab = (sv - lo) * L
            i1 = id1_v[pl.ds(i * L, L)] + s
            i2 = id2_v[pl.ds(i * L, L)] + s
            plsc.store_compressed(ms_v.at[pl.ds(p, L)], iab, mask=m)
            plsc.store_compressed(m1_v.at[pl.ds(p, L)], i1, mask=m)
            plsc.store_compressed(m2_v.at[pl.ds(p, L)], i2, mask=m)
            return p + jnp.sum(m.astype(jnp.int32))

        p = lax.fori_loop(0, CH2 // L, scan_body, 0)

        @pl.when(ch + 1 < NCH)
        def _():
            issue_chunk(ch + 1)

        for t in range(SB // L):
            ms_v[pl.ds(p + t * L, L)] = _splat_i32(NPR * L)
            m1_v[pl.ds(p + t * L, L)] = _splat_i32(0)
            m2_v[pl.ds(p + t * L, L)] = _splat_i32(0)

        nsb = (p + SB - 1) // SB

        @pl.when(nsb > 0)
        def _():
            issue_sb(0, r1a_v, r2a_v, semA)

        def pair_body(tt, _):
            t0 = tt * 2

            @pl.when(t0 < nsb)
            def _():
                @pl.when(t0 + 1 < nsb)
                def _():
                    issue_sb(t0 + 1, r1b_v, r2b_v, semB)

                wait_sb(r1a_v, r2a_v, semA)
                rmw_sb(t0, r1a_v, r2a_v)

            @pl.when(t0 + 1 < nsb)
            def _():
                @pl.when(t0 + 2 < nsb)
                def _():
                    issue_sb(t0 + 2, r1a_v, r2a_v, semA)

                wait_sb(r1b_v, r2b_v, semB)
                rmw_sb(t0 + 1, r1b_v, r2b_v)

            return 0

        lax.fori_loop(0, (nsb + 1) // 2, pair_body, 0)
        return 0

    lax.fori_loop(0, NCH, chunk_body, 0)
    pltpu.sync_copy(acc_v.at[pl.ds(0, NPR * L)],
                    out_hbm.at[pl.ds((s * N + lo) * L, NPR * L)])


def _mm_kernel(x_ref, w_ref, o_ref):
    o_ref[...] = jnp.dot(x_ref[...], w_ref[...],
                         preferred_element_type=jnp.float32)


def _matmul(x, w, bn):
    n = x.shape[0]
    return pl.pallas_call(
        _mm_kernel,
        grid=(n // bn,),
        in_specs=[pl.BlockSpec((bn, x.shape[1]), lambda i: (i, 0)),
                  pl.BlockSpec((x.shape[1], w.shape[1]), lambda i: (0, 0))],
        out_specs=pl.BlockSpec((bn, w.shape[1]), lambda i: (i, 0)),
        out_shape=jax.ShapeDtypeStruct((n, w.shape[1]), jnp.float32),
    )(x, w)


def _mlp_kernel(p_ref, w1_ref, w2_ref, o_ref):
    p = p_ref[...]
    h = jnp.maximum(jnp.dot(p, w1_ref[...], preferred_element_type=jnp.float32), 0.0)
    o_ref[...] = jnp.dot(h, w2_ref[...], preferred_element_type=jnp.float32) + p


def _mlp(pooled, w1t, w2t, bn):
    n = pooled.shape[0]
    return pl.pallas_call(
        _mlp_kernel,
        grid=(n // bn,),
        in_specs=[pl.BlockSpec((bn, OUT), lambda i: (i, 0)),
                  pl.BlockSpec((OUT, 2 * OUT), lambda i: (0, 0)),
                  pl.BlockSpec((2 * OUT, OUT), lambda i: (0, 0))],
        out_specs=pl.BlockSpec((bn, OUT), lambda i: (i, 0)),
        out_shape=jax.ShapeDtypeStruct((n, OUT), jnp.float32),
    )(pooled, w1t, w2t)


def kernel(x, pos, edge_index, expert_weights, gate_w, gate_b, w1, w2):
    gwi = lax.bitcast_convert_type(gate_w, jnp.int32)
    gwi = (gwi + 0x7FFF + (lax.shift_right_logical(gwi, 16) & 1)) & jnp.int32(-65536)
    gwr = lax.bitcast_convert_type(gwi, jnp.float32)
    gp = jnp.concatenate([gwr.reshape(-1), gate_b,
                          jnp.zeros((104,), jnp.float32)])
    src_a, id1_a, id2_a = _route(pos[:, :DIM].reshape(-1),
                                 edge_index.reshape(-1), gp)

    w_fused = jnp.transpose(expert_weights, (1, 0, 2)).reshape(IN, K * OUT)
    y = _matmul(x, w_fused, 1000)

    pooled_t = _segmax(src_a, id1_a, id2_a, y.reshape(N * K * NSLICE, L))
    pooled = jnp.transpose(pooled_t.reshape(NSLICE, N, L),
                           (1, 0, 2)).reshape(N, OUT)
    return _mlp(pooled, w1.T, w2.T, 1000)
